# baseline (speedup 1.0000x reference)
import functools

import jax
import jax.numpy as jnp
from jax import lax
from jax.experimental import pallas as pl
from jax.experimental.pallas import tpu as pltpu
from jax.experimental.pallas import tpu_sc as plsc

D = 32
NC = 2
NS = 16
NW = NC * NS
CH = 1024
NB = 3
TU = 128


BU = 6


@functools.lru_cache(maxsize=None)
def _make_transpose(V: int):
    n_units = (V + TU - 1) // TU
    v_pad = n_units * TU
    n_full = V // TU
    assert n_full % BU == 0
    n_blk = n_full // BU
    cols = BU * TU
    mesh = plsc.VectorSubcoreMesh(core_axis_name="c", subcore_axis_name="s")

    @functools.partial(
        pl.kernel,
        mesh=mesh,
        out_type=jax.ShapeDtypeStruct((v_pad * D,), jnp.float32),
        compiler_params=pltpu.CompilerParams(needs_layout_passes=False),
        scratch_types=[
            [pltpu.VMEM((D, cols + 1), jnp.float32) for _ in range(2)],
            [pltpu.VMEM((cols * D,), jnp.float32) for _ in range(2)],
            [pltpu.SemaphoreType.DMA for _ in range(2)],
            [pltpu.SemaphoreType.DMA for _ in range(2)],
        ],
    )
    def transpose_kernel(tt_hbm, tail_hbm, out_hbm, bins, bouts, gs, ss):
        wid = lax.axis_index("s") * NC + lax.axis_index("c")
        lo = jax.lax.iota(jnp.int32, 16)
        n_tail = V - n_full * TU
        nk = (n_blk + NW - 1) // NW

        def blk_of(k):
            return wid + k * NW

        def copy_in(k, t):
            for i in range(BU):
                pltpu.async_copy(
                    tt_hbm.at[:, pl.ds((blk_of(k) * BU + i) * TU, TU)],
                    bins[t].at[:, pl.ds(i * TU, TU)],
                    gs[t],
                )

        def wait_in(t):
            pltpu.make_async_copy(
                tt_hbm.at[:, pl.ds(0, cols)], bins[t].at[:, pl.ds(0, cols)], gs[t]
            ).wait()

        def wait_out(t):
            pltpu.make_async_copy(
                bouts[t], out_hbm.at[pl.ds(0, cols * D)], ss[t]
            ).wait()

        def compute(t):
            @plsc.parallel_loop(0, cols, step=4, unroll=4)
            def _cbody(c0):
                for cc in range(4):
                    col = jnp.zeros((16,), jnp.int32) + (c0 + cc)
                    v0 = plsc.load_gather(bins[t], [lo, col])
                    v1 = plsc.load_gather(bins[t], [lo + 16, col])
                    bouts[t][pl.ds((c0 + cc) * D, 16)] = v0
                    bouts[t][pl.ds((c0 + cc) * D + 16, 16)] = v1

        def step(k, t):
            @pl.when(blk_of(k) < n_blk)
            def _():
                @pl.when((k + 1 < nk) & (blk_of(k + 1) < n_blk))
                def _():
                    copy_in(k + 1, 1 - t)

                wait_in(t)

                @pl.when(k >= 2)
                def _():
                    wait_out(t)

                compute(t)
                pltpu.async_copy(
                    bouts[t],
                    out_hbm.at[pl.ds(blk_of(k) * (cols * D), cols * D)],
                    ss[t],
                )

        @pl.when(blk_of(0) < n_blk)
        def _():
            copy_in(0, 0)

        def body(j, _):
            for t in range(2):
                k = j * 2 + t

                @pl.when(k < nk)
                def _():
                    step(k, t)

            return 0

        lax.fori_loop(0, (nk + 1) // 2, body, 0)
        for t in range(2):
            @pl.when(blk_of(t) < n_blk)
            def _():
                wait_out(t)
        if n_tail:
            @pl.when(wid == 0)
            def _():
                pltpu.sync_copy(
                    tail_hbm, out_hbm.at[pl.ds(n_full * (TU * D), n_tail * D)]
                )

    return transpose_kernel


@functools.lru_cache(maxsize=None)
def _make_gather(B: int, V: int):
    assert B % (NW * CH) == 0
    b_per_w = B // NW
    n_ch = b_per_w // CH
    mesh = plsc.VectorSubcoreMesh(core_axis_name="c", subcore_axis_name="s")

    @functools.partial(
        pl.kernel,
        mesh=mesh,
        out_type=jax.ShapeDtypeStruct((B, D), jnp.float32),
        compiler_params=pltpu.CompilerParams(use_tc_tiling_on_sc=False),
        scratch_types=[
            pltpu.VMEM((b_per_w,), jnp.int32),
            [pltpu.VMEM((CH, D), jnp.float32) for _ in range(NB)],
            [pltpu.SemaphoreType.DMA for _ in range(NB)],
            [pltpu.SemaphoreType.DMA for _ in range(NB)],
        ],
    )
    def gather_kernel(idx_hbm, table_hbm, out_hbm, idx_v, bufs, gsems, ssems):
        wid = lax.axis_index("s") * NC + lax.axis_index("c")
        base = wid * b_per_w
        pltpu.sync_copy(idx_hbm.at[pl.ds(base, b_per_w)], idx_v)

        def gather(j, b):
            return pltpu.async_copy(
                table_hbm.at[idx_v.at[pl.ds(j * CH, CH)]], bufs[b], gsems[b]
            )

        g = [None] * NB
        st = [None] * NB
        for b in range(min(NB - 1, n_ch)):
            g[b] = gather(b, b)
        for j in range(n_ch):
            b = j % NB
            jj = j + NB - 1
            if jj < n_ch:
                bb = jj % NB
                if jj >= NB:
                    st[bb].wait()
                g[bb] = gather(jj, bb)
            g[b].wait()
            st[b] = pltpu.async_copy(
                bufs[b], out_hbm.at[pl.ds(base + j * CH, CH)], ssems[b]
            )
        for j in range(max(0, n_ch - NB), n_ch):
            st[j % NB].wait()

    return gather_kernel


def kernel(inputs, embedding_matrix):
    B, F = inputs.shape
    V, d = embedding_matrix.shape
    n = B * F
    idx = inputs.reshape(n).astype(jnp.int32)
    v_pad = ((V + TU - 1) // TU) * TU
    tail = embedding_matrix[(V // TU) * TU:].reshape(-1)
    table_lin = _make_transpose(V)(embedding_matrix.T, tail).reshape(v_pad, d)
    out = _make_gather(n, V)(idx, table_lin)
    return out.reshape(B, F, d)


# speedup vs baseline: 1.0061x; 1.0061x over previous
import functools

import jax
import jax.numpy as jnp
from jax import lax
from jax.experimental import pallas as pl
from jax.experimental.pallas import tpu as pltpu
from jax.experimental.pallas import tpu_sc as plsc

D = 32
NC = 2
NS = 16
NW = NC * NS
CH = 1024
NB = 3
TU = 128


BU = 6


@functools.lru_cache(maxsize=None)
def _make_transpose(V: int):
    n_units = (V + TU - 1) // TU
    v_pad = n_units * TU
    n_full = V // TU
    assert n_full % BU == 0
    n_blk = n_full // BU
    cols = BU * TU
    mesh = plsc.VectorSubcoreMesh(core_axis_name="c", subcore_axis_name="s")

    @functools.partial(
        pl.kernel,
        mesh=mesh,
        out_type=jax.ShapeDtypeStruct((v_pad * D,), jnp.float32),
        compiler_params=pltpu.CompilerParams(needs_layout_passes=False),
        scratch_types=[
            [pltpu.VMEM((D, cols + 1), jnp.float32) for _ in range(2)],
            [pltpu.VMEM((cols * D,), jnp.float32) for _ in range(2)],
            [pltpu.SemaphoreType.DMA for _ in range(2)],
            [pltpu.SemaphoreType.DMA for _ in range(2)],
        ],
    )
    def transpose_kernel(tt_hbm, tail_hbm, out_hbm, bins, bouts, gs, ss):
        wid = lax.axis_index("s") * NC + lax.axis_index("c")
        lo = jax.lax.iota(jnp.int32, 16)
        n_tail = V - n_full * TU
        nk = (n_blk + NW - 1) // NW

        def blk_of(k):
            return wid + k * NW

        def copy_in(k, t):
            for i in range(BU):
                pltpu.async_copy(
                    tt_hbm.at[:, pl.ds((blk_of(k) * BU + i) * TU, TU)],
                    bins[t].at[:, pl.ds(i * TU, TU)],
                    gs[t],
                )

        def wait_in(t):
            pltpu.make_async_copy(
                tt_hbm.at[:, pl.ds(0, cols)], bins[t].at[:, pl.ds(0, cols)], gs[t]
            ).wait()

        def wait_out(t):
            pltpu.make_async_copy(
                bouts[t], out_hbm.at[pl.ds(0, cols * D)], ss[t]
            ).wait()

        def compute(t):
            @plsc.parallel_loop(0, cols, step=4, unroll=8)
            def _cbody(c0):
                for cc in range(4):
                    col = jnp.zeros((16,), jnp.int32) + (c0 + cc)
                    v0 = plsc.load_gather(bins[t], [lo, col])
                    v1 = plsc.load_gather(bins[t], [lo + 16, col])
                    bouts[t][pl.ds((c0 + cc) * D, 16)] = v0
                    bouts[t][pl.ds((c0 + cc) * D + 16, 16)] = v1

        def step(k, t):
            @pl.when(blk_of(k) < n_blk)
            def _():
                @pl.when((k + 1 < nk) & (blk_of(k + 1) < n_blk))
                def _():
                    copy_in(k + 1, 1 - t)

                wait_in(t)

                @pl.when(k >= 2)
                def _():
                    wait_out(t)

                compute(t)
                pltpu.async_copy(
                    bouts[t],
                    out_hbm.at[pl.ds(blk_of(k) * (cols * D), cols * D)],
                    ss[t],
                )

        @pl.when(blk_of(0) < n_blk)
        def _():
            copy_in(0, 0)

        def body(j, _):
            for t in range(2):
                k = j * 2 + t

                @pl.when(k < nk)
                def _():
                    step(k, t)

            return 0

        lax.fori_loop(0, (nk + 1) // 2, body, 0)
        for t in range(2):
            @pl.when(blk_of(t) < n_blk)
            def _():
                wait_out(t)
        if n_tail:
            @pl.when(wid == 0)
            def _():
                pltpu.sync_copy(
                    tail_hbm, out_hbm.at[pl.ds(n_full * (TU * D), n_tail * D)]
                )

    return transpose_kernel


@functools.lru_cache(maxsize=None)
def _make_gather(B: int, V: int):
    assert B % (NW * CH) == 0
    b_per_w = B // NW
    n_ch = b_per_w // CH
    mesh = plsc.VectorSubcoreMesh(core_axis_name="c", subcore_axis_name="s")

    @functools.partial(
        pl.kernel,
        mesh=mesh,
        out_type=jax.ShapeDtypeStruct((B, D), jnp.float32),
        compiler_params=pltpu.CompilerParams(use_tc_tiling_on_sc=False),
        scratch_types=[
            pltpu.VMEM((b_per_w,), jnp.int32),
            [pltpu.VMEM((CH, D), jnp.float32) for _ in range(NB)],
            [pltpu.SemaphoreType.DMA for _ in range(NB)],
            [pltpu.SemaphoreType.DMA for _ in range(NB)],
        ],
    )
    def gather_kernel(idx_hbm, table_hbm, out_hbm, idx_v, bufs, gsems, ssems):
        wid = lax.axis_index("s") * NC + lax.axis_index("c")
        base = wid * b_per_w
        pltpu.sync_copy(idx_hbm.at[pl.ds(base, b_per_w)], idx_v)

        def gather(j, b):
            return pltpu.async_copy(
                table_hbm.at[idx_v.at[pl.ds(j * CH, CH)]], bufs[b], gsems[b]
            )

        g = [None] * NB
        st = [None] * NB
        for b in range(min(NB - 1, n_ch)):
            g[b] = gather(b, b)
        for j in range(n_ch):
            b = j % NB
            jj = j + NB - 1
            if jj < n_ch:
                bb = jj % NB
                if jj >= NB:
                    st[bb].wait()
                g[bb] = gather(jj, bb)
            g[b].wait()
            st[b] = pltpu.async_copy(
                bufs[b], out_hbm.at[pl.ds(base + j * CH, CH)], ssems[b]
            )
        for j in range(max(0, n_ch - NB), n_ch):
            st[j % NB].wait()

    return gather_kernel


def kernel(inputs, embedding_matrix):
    B, F = inputs.shape
    V, d = embedding_matrix.shape
    n = B * F
    idx = inputs.reshape(n).astype(jnp.int32)
    v_pad = ((V + TU - 1) // TU) * TU
    tail = embedding_matrix[(V // TU) * TU:].reshape(-1)
    table_lin = _make_transpose(V)(embedding_matrix.T, tail).reshape(v_pad, d)
    out = _make_gather(n, V)(idx, table_lin)
    return out.reshape(B, F, d)


# speedup vs baseline: 1.0527x; 1.0463x over previous
import functools

import jax
import jax.numpy as jnp
from jax import lax
from jax.experimental import pallas as pl
from jax.experimental.pallas import tpu as pltpu
from jax.experimental.pallas import tpu_sc as plsc

D = 32
NC = 2
NS = 16
NW = NC * NS
CH = 1024
NB = 3
TU = 128


BU = 6


@functools.lru_cache(maxsize=None)
def _make_transpose(V: int):
    n_units = (V + TU - 1) // TU
    v_pad = n_units * TU
    n_full = V // TU
    assert n_full % BU == 0
    n_blk = n_full // BU
    cols = BU * TU
    mesh = plsc.VectorSubcoreMesh(core_axis_name="c", subcore_axis_name="s")

    @functools.partial(
        pl.kernel,
        mesh=mesh,
        out_type=jax.ShapeDtypeStruct((v_pad * D,), jnp.float32),
        compiler_params=pltpu.CompilerParams(needs_layout_passes=False),
        scratch_types=[
            [pltpu.VMEM((D, cols + 1), jnp.float32) for _ in range(2)],
            [pltpu.VMEM((cols * D,), jnp.float32) for _ in range(2)],
            [pltpu.SemaphoreType.DMA for _ in range(2)],
            [pltpu.SemaphoreType.DMA for _ in range(2)],
        ],
    )
    def transpose_kernel(tt_hbm, tail_hbm, out_hbm, bins, bouts, gs, ss):
        wid = lax.axis_index("s") * NC + lax.axis_index("c")
        lo = jax.lax.iota(jnp.int32, 16)
        n_tail = V - n_full * TU
        nk = (n_blk + NW - 1) // NW

        def blk_of(k):
            return wid + k * NW

        def copy_in(k, t):
            for i in range(BU):
                pltpu.async_copy(
                    tt_hbm.at[:, pl.ds((blk_of(k) * BU + i) * TU, TU)],
                    bins[t].at[:, pl.ds(i * TU, TU)],
                    gs[t],
                )

        def wait_in(t):
            pltpu.make_async_copy(
                tt_hbm.at[:, pl.ds(0, cols)], bins[t].at[:, pl.ds(0, cols)], gs[t]
            ).wait()

        def wait_out(t):
            pltpu.make_async_copy(
                bouts[t], out_hbm.at[pl.ds(0, cols * D)], ss[t]
            ).wait()

        base = jax.lax.iota(jnp.int32, 16) * D

        def compute(t):
            @plsc.parallel_loop(0, cols, step=16, unroll=2)
            def _cbody(c0):
                for d in range(D):
                    v = bins[t][d, pl.ds(c0, 16)]
                    plsc.store_scatter(bouts[t], [base + (c0 * D + d)], v)

        def step(k, t):
            @pl.when(blk_of(k) < n_blk)
            def _():
                @pl.when((k + 1 < nk) & (blk_of(k + 1) < n_blk))
                def _():
                    copy_in(k + 1, 1 - t)

                wait_in(t)

                @pl.when(k >= 2)
                def _():
                    wait_out(t)

                compute(t)
                pltpu.async_copy(
                    bouts[t],
                    out_hbm.at[pl.ds(blk_of(k) * (cols * D), cols * D)],
                    ss[t],
                )

        @pl.when(blk_of(0) < n_blk)
        def _():
            copy_in(0, 0)

        def body(j, _):
            for t in range(2):
                k = j * 2 + t

                @pl.when(k < nk)
                def _():
                    step(k, t)

            return 0

        lax.fori_loop(0, (nk + 1) // 2, body, 0)
        for t in range(2):
            @pl.when(blk_of(t) < n_blk)
            def _():
                wait_out(t)
        if n_tail:
            @pl.when(wid == 0)
            def _():
                pltpu.sync_copy(
                    tail_hbm, out_hbm.at[pl.ds(n_full * (TU * D), n_tail * D)]
                )

    return transpose_kernel


@functools.lru_cache(maxsize=None)
def _make_gather(B: int, V: int):
    assert B % (NW * CH) == 0
    b_per_w = B // NW
    n_ch = b_per_w // CH
    mesh = plsc.VectorSubcoreMesh(core_axis_name="c", subcore_axis_name="s")

    @functools.partial(
        pl.kernel,
        mesh=mesh,
        out_type=jax.ShapeDtypeStruct((B, D), jnp.float32),
        compiler_params=pltpu.CompilerParams(use_tc_tiling_on_sc=False),
        scratch_types=[
            pltpu.VMEM((b_per_w,), jnp.int32),
            [pltpu.VMEM((CH, D), jnp.float32) for _ in range(NB)],
            [pltpu.SemaphoreType.DMA for _ in range(NB)],
            [pltpu.SemaphoreType.DMA for _ in range(NB)],
        ],
    )
    def gather_kernel(idx_hbm, table_hbm, out_hbm, idx_v, bufs, gsems, ssems):
        wid = lax.axis_index("s") * NC + lax.axis_index("c")
        base = wid * b_per_w
        pltpu.sync_copy(idx_hbm.at[pl.ds(base, b_per_w)], idx_v)

        def gather(j, b):
            return pltpu.async_copy(
                table_hbm.at[idx_v.at[pl.ds(j * CH, CH)]], bufs[b], gsems[b]
            )

        g = [None] * NB
        st = [None] * NB
        for b in range(min(NB - 1, n_ch)):
            g[b] = gather(b, b)
        for j in range(n_ch):
            b = j % NB
            jj = j + NB - 1
            if jj < n_ch:
                bb = jj % NB
                if jj >= NB:
                    st[bb].wait()
                g[bb] = gather(jj, bb)
            g[b].wait()
            st[b] = pltpu.async_copy(
                bufs[b], out_hbm.at[pl.ds(base + j * CH, CH)], ssems[b]
            )
        for j in range(max(0, n_ch - NB), n_ch):
            st[j % NB].wait()

    return gather_kernel


def kernel(inputs, embedding_matrix):
    B, F = inputs.shape
    V, d = embedding_matrix.shape
    n = B * F
    idx = inputs.reshape(n).astype(jnp.int32)
    v_pad = ((V + TU - 1) // TU) * TU
    tail = embedding_matrix[(V // TU) * TU:].reshape(-1)
    table_lin = _make_transpose(V)(embedding_matrix.T, tail).reshape(v_pad, d)
    out = _make_gather(n, V)(idx, table_lin)
    return out.reshape(B, F, d)


# speedup vs baseline: 2.0101x; 1.9094x over previous
import functools

import jax
import jax.numpy as jnp
from jax import lax
from jax.experimental import pallas as pl
from jax.experimental.pallas import tpu as pltpu
from jax.experimental.pallas import tpu_sc as plsc

D = 32
NC = 2
NS = 16
NW = NC * NS
CH = 1024
NB = 3
TU = 128


BU = 6


@functools.lru_cache(maxsize=None)
def _make_transpose(V: int):
    n_units = (V + TU - 1) // TU
    v_pad = n_units * TU
    n_full = V // TU
    assert n_full % BU == 0
    n_blk = n_full // BU
    cols = BU * TU
    mesh = plsc.VectorSubcoreMesh(core_axis_name="c", subcore_axis_name="s")

    @functools.partial(
        pl.kernel,
        mesh=mesh,
        out_type=jax.ShapeDtypeStruct((v_pad * D,), jnp.float32),
        compiler_params=pltpu.CompilerParams(needs_layout_passes=False),
        scratch_types=[
            [pltpu.VMEM((D, cols + 2), jnp.float32) for _ in range(2)],
            [pltpu.VMEM((cols * D,), jnp.float32) for _ in range(2)],
            [pltpu.SemaphoreType.DMA for _ in range(2)],
            [pltpu.SemaphoreType.DMA for _ in range(2)],
        ],
    )
    def transpose_kernel(tt_hbm, tail_hbm, out_hbm, bins, bouts, gs, ss):
        wid = lax.axis_index("s") * NC + lax.axis_index("c")
        lo = jax.lax.iota(jnp.int32, 16)
        n_tail = V - n_full * TU
        nk = (n_blk + NW - 1) // NW

        def blk_of(k):
            return wid + k * NW

        def copy_in(k, t):
            for i in range(BU):
                pltpu.async_copy(
                    tt_hbm.at[:, pl.ds((blk_of(k) * BU + i) * TU, TU)],
                    bins[t].at[:, pl.ds(i * TU, TU)],
                    gs[t],
                )

        def wait_in(t):
            pltpu.make_async_copy(
                tt_hbm.at[:, pl.ds(0, cols)], bins[t].at[:, pl.ds(0, cols)], gs[t]
            ).wait()

        def wait_out(t):
            pltpu.make_async_copy(
                bouts[t], out_hbm.at[pl.ds(0, cols * D)], ss[t]
            ).wait()

        i16 = jax.lax.iota(jnp.int32, 16)
        rows_w = [(i16 + w) & 31 for w in range(D)]

        def compute(t):
            @plsc.parallel_loop(0, cols, step=16, unroll=2)
            def _cbody(c0):
                colv = i16 + c0
                colD = colv * D
                for w in range(D):
                    v = plsc.load_gather(bins[t], [rows_w[w], colv])
                    plsc.store_scatter(bouts[t], [colD + rows_w[w]], v)

        def step(k, t):
            @pl.when(blk_of(k) < n_blk)
            def _():
                @pl.when((k + 1 < nk) & (blk_of(k + 1) < n_blk))
                def _():
                    copy_in(k + 1, 1 - t)

                wait_in(t)

                @pl.when(k >= 2)
                def _():
                    wait_out(t)

                compute(t)
                pltpu.async_copy(
                    bouts[t],
                    out_hbm.at[pl.ds(blk_of(k) * (cols * D), cols * D)],
                    ss[t],
                )

        @pl.when(blk_of(0) < n_blk)
        def _():
            copy_in(0, 0)

        def body(j, _):
            for t in range(2):
                k = j * 2 + t

                @pl.when(k < nk)
                def _():
                    step(k, t)

            return 0

        lax.fori_loop(0, (nk + 1) // 2, body, 0)
        for t in range(2):
            @pl.when(blk_of(t) < n_blk)
            def _():
                wait_out(t)
        if n_tail:
            @pl.when(wid == 0)
            def _():
                pltpu.sync_copy(
                    tail_hbm, out_hbm.at[pl.ds(n_full * (TU * D), n_tail * D)]
                )

    return transpose_kernel


@functools.lru_cache(maxsize=None)
def _make_gather(B: int, V: int):
    assert B % (NW * CH) == 0
    b_per_w = B // NW
    n_ch = b_per_w // CH
    mesh = plsc.VectorSubcoreMesh(core_axis_name="c", subcore_axis_name="s")

    @functools.partial(
        pl.kernel,
        mesh=mesh,
        out_type=jax.ShapeDtypeStruct((B, D), jnp.float32),
        compiler_params=pltpu.CompilerParams(use_tc_tiling_on_sc=False),
        scratch_types=[
            pltpu.VMEM((b_per_w,), jnp.int32),
            [pltpu.VMEM((CH, D), jnp.float32) for _ in range(NB)],
            [pltpu.SemaphoreType.DMA for _ in range(NB)],
            [pltpu.SemaphoreType.DMA for _ in range(NB)],
        ],
    )
    def gather_kernel(idx_hbm, table_hbm, out_hbm, idx_v, bufs, gsems, ssems):
        wid = lax.axis_index("s") * NC + lax.axis_index("c")
        base = wid * b_per_w
        pltpu.sync_copy(idx_hbm.at[pl.ds(base, b_per_w)], idx_v)

        def gather(j, b):
            return pltpu.async_copy(
                table_hbm.at[idx_v.at[pl.ds(j * CH, CH)]], bufs[b], gsems[b]
            )

        g = [None] * NB
        st = [None] * NB
        for b in range(min(NB - 1, n_ch)):
            g[b] = gather(b, b)
        for j in range(n_ch):
            b = j % NB
            jj = j + NB - 1
            if jj < n_ch:
                bb = jj % NB
                if jj >= NB:
                    st[bb].wait()
                g[bb] = gather(jj, bb)
            g[b].wait()
            st[b] = pltpu.async_copy(
                bufs[b], out_hbm.at[pl.ds(base + j * CH, CH)], ssems[b]
            )
        for j in range(max(0, n_ch - NB), n_ch):
            st[j % NB].wait()

    return gather_kernel


def kernel(inputs, embedding_matrix):
    B, F = inputs.shape
    V, d = embedding_matrix.shape
    n = B * F
    idx = inputs.reshape(n).astype(jnp.int32)
    v_pad = ((V + TU - 1) // TU) * TU
    tail = embedding_matrix[(V // TU) * TU:].reshape(-1)
    table_lin = _make_transpose(V)(embedding_matrix.T, tail).reshape(v_pad, d)
    out = _make_gather(n, V)(idx, table_lin)
    return out.reshape(B, F, d)
